# baseline (speedup 1.0000x reference)
import functools

import jax
import jax.numpy as jnp
from jax.experimental import pallas as pl
from jax.experimental.pallas import tpu as pltpu

_CAP = 2
_BS = 512


def _routing_body(x_ref, gw_ref, g_ref, aff_ref, *, K, S, E, B, NC):
    ci = pl.program_id(0)
    SC = S // NC
    for bi in range(B):
        logits = jnp.dot(x_ref[bi], gw_ref[...],
                         preferred_element_type=jnp.float32)
        m = jnp.max(logits, axis=-1, keepdims=True)
        ex = jnp.exp(logits - m)
        aff_c = ex / jnp.sum(ex, axis=-1, keepdims=True)
        aff_ref[pl.ds(ci * SC, SC), bi * E:(bi + 1) * E] = aff_c

    @pl.when(ci == NC - 1)
    def _bisect():
        _routing_epilogue(aff_ref, g_ref, K=K, S=S, E=E, B=B)


def _routing_epilogue(aff_ref, g_ref, *, K, S, E, B):
    aff = aff_ref[...]
    ai = jax.lax.bitcast_convert_type(aff, jnp.int32)
    BE = B * E

    def vstep(_, carry):
        lo, hi = carry
        mid = lo + (hi - lo) // 2
        cnt = jnp.sum((ai >= mid).astype(jnp.int32), axis=0, keepdims=True)
        ge = cnt >= K
        return jnp.where(ge, mid, lo), jnp.where(ge, hi, mid)

    lo0 = jnp.zeros((1, BE), jnp.int32)
    hi0 = jnp.full((1, BE), jnp.int32(0x7F800000))
    T, _ = jax.lax.fori_loop(0, 31, vstep, (lo0, hi0))

    gt = ai > T
    tie = ai == T
    n_gt = jnp.sum(gt.astype(jnp.int32), axis=0, keepdims=True)
    d = K - n_gt

    idx = jax.lax.broadcasted_iota(jnp.int32, (S, BE), 0)

    def istep(_, carry):
        lo2, hi2 = carry
        mid = lo2 + (hi2 - lo2) // 2
        cnt = jnp.sum((tie & (idx <= mid)).astype(jnp.int32),
                      axis=0, keepdims=True)
        ok = cnt >= d
        return jnp.where(ok, lo2, mid), jnp.where(ok, mid, hi2)

    lo2 = jnp.full((1, BE), -1, jnp.int32)
    hi2 = jnp.full((1, BE), S - 1, jnp.int32)
    _, M = jax.lax.fori_loop(0, 12, istep, (lo2, hi2))

    mask = gt | (tie & (idx <= M))
    g = aff * mask.astype(jnp.float32)
    for bi in range(B):
        g_ref[bi] = g[:, bi * E:(bi + 1) * E]


def _expert_body(x_ref, w_ref, b_ref, g_ref, out_ref, wb_ref, *, E):
    si = pl.program_id(0) * pl.num_programs(1) + pl.program_id(1)

    @pl.when(si == 0)
    def _convert_weights():
        for e in range(E):
            wb_ref[e] = w_ref[e].astype(jnp.bfloat16)

    xx = x_ref[0].astype(jnp.bfloat16)
    gg = g_ref[0]
    acc = None
    for e in range(E):
        y = jnp.dot(xx, wb_ref[e],
                    preferred_element_type=jnp.float32)
        y = y + b_ref[e:e + 1, :]
        c = gg[:, e:e + 1] * y
        acc = c if acc is None else acc + c
    out_ref[0] = acc


@jax.jit
def kernel(x, gate_weight, W, b):
    B, S, D = x.shape
    E = gate_weight.shape[1]
    K = (S * _CAP) // E

    NC = 8
    g = pl.pallas_call(
        functools.partial(_routing_body, K=K, S=S, E=E, B=B, NC=NC),
        grid=(NC,),
        in_specs=[
            pl.BlockSpec((B, S // NC, D), lambda i: (0, i, 0)),
            pl.BlockSpec((D, E), lambda i: (0, 0)),
        ],
        out_specs=pl.BlockSpec((B, S, E), lambda i: (0, 0, 0)),
        out_shape=jax.ShapeDtypeStruct((B, S, E), jnp.float32),
        scratch_shapes=[pltpu.VMEM((S, B * E), jnp.float32)],
    )(x, gate_weight)

    return pl.pallas_call(
        functools.partial(_expert_body, E=E),
        grid=(B, S // _BS),
        in_specs=[
            pl.BlockSpec((1, _BS, D), lambda bi, si: (bi, si, 0)),
            pl.BlockSpec((E, D, D), lambda bi, si: (0, 0, 0)),
            pl.BlockSpec((E, D), lambda bi, si: (0, 0)),
            pl.BlockSpec((1, _BS, E), lambda bi, si: (bi, si, 0)),
        ],
        out_specs=pl.BlockSpec((1, _BS, D), lambda bi, si: (bi, si, 0)),
        out_shape=jax.ShapeDtypeStruct((B, S, D), jnp.float32),
        scratch_shapes=[pltpu.VMEM((E, D, D), jnp.bfloat16)],
    )(x, W, b, g)


# speedup vs baseline: 1.0170x; 1.0170x over previous
import functools

import jax
import jax.numpy as jnp
from jax.experimental import pallas as pl
from jax.experimental.pallas import tpu as pltpu

_CAP = 2
_BS = 512


def _routing_body(x_ref, gw_ref, g_ref, aff_ref, *, K, S, E, B, NC):
    ci = pl.program_id(0)
    SC = S // NC
    for bi in range(B):
        logits = jnp.dot(x_ref[bi], gw_ref[...],
                         preferred_element_type=jnp.float32)
        m = jnp.max(logits, axis=-1, keepdims=True)
        ex = jnp.exp(logits - m)
        aff_c = ex / jnp.sum(ex, axis=-1, keepdims=True)
        aff_ref[pl.ds(ci * SC, SC), bi * E:(bi + 1) * E] = aff_c

    @pl.when(ci == NC - 1)
    def _bisect():
        _routing_epilogue(aff_ref, g_ref, K=K, S=S, E=E, B=B)


def _routing_epilogue(aff_ref, g_ref, *, K, S, E, B):
    aff = aff_ref[...]
    ai = jax.lax.bitcast_convert_type(aff, jnp.int32)
    BE = B * E

    def vstep(_, carry):
        lo, hi = carry
        mid = lo + (hi - lo) // 2
        cnt = jnp.sum((ai >= mid).astype(jnp.int32), axis=0, keepdims=True)
        ge = cnt >= K
        return jnp.where(ge, mid, lo), jnp.where(ge, hi, mid)

    lo0 = jnp.zeros((1, BE), jnp.int32)
    hi0 = jnp.full((1, BE), jnp.int32(0x7F800000))
    T, _ = jax.lax.fori_loop(0, 31, vstep, (lo0, hi0))

    gt = ai > T
    tie = ai == T
    n_gt = jnp.sum(gt.astype(jnp.int32), axis=0, keepdims=True)
    d = K - n_gt

    idx = jax.lax.broadcasted_iota(jnp.int32, (S, BE), 0)

    def istep(_, carry):
        lo2, hi2 = carry
        mid = lo2 + (hi2 - lo2) // 2
        cnt = jnp.sum((tie & (idx <= mid)).astype(jnp.int32),
                      axis=0, keepdims=True)
        ok = cnt >= d
        return jnp.where(ok, lo2, mid), jnp.where(ok, mid, hi2)

    lo2 = jnp.full((1, BE), -1, jnp.int32)
    hi2 = jnp.full((1, BE), S - 1, jnp.int32)
    _, M = jax.lax.fori_loop(0, 12, istep, (lo2, hi2))

    mask = gt | (tie & (idx <= M))
    g = aff * mask.astype(jnp.float32)
    for bi in range(B):
        g_ref[bi] = g[:, bi * E:(bi + 1) * E]


def _expert_body(x_ref, w_ref, b_ref, g_ref, out_ref, wb_ref, *, E):
    si = pl.program_id(0) * pl.num_programs(1) + pl.program_id(1)

    @pl.when(si == 0)
    def _convert_weights():
        for e in range(E):
            wb_ref[e] = w_ref[e].astype(jnp.bfloat16)

    xx = x_ref[0].astype(jnp.bfloat16)
    gg = g_ref[0]
    acc = None
    for e in range(E):
        y = jnp.dot(xx, wb_ref[e],
                    preferred_element_type=jnp.float32)
        y = y + b_ref[e:e + 1, :]
        c = gg[:, e:e + 1] * y
        acc = c if acc is None else acc + c
    out_ref[0] = acc


@jax.jit
def kernel(x, gate_weight, W, b):
    B, S, D = x.shape
    E = gate_weight.shape[1]
    K = (S * _CAP) // E

    NC = 2
    g = pl.pallas_call(
        functools.partial(_routing_body, K=K, S=S, E=E, B=B, NC=NC),
        grid=(NC,),
        in_specs=[
            pl.BlockSpec((B, S // NC, D), lambda i: (0, i, 0)),
            pl.BlockSpec((D, E), lambda i: (0, 0)),
        ],
        out_specs=pl.BlockSpec((B, S, E), lambda i: (0, 0, 0)),
        out_shape=jax.ShapeDtypeStruct((B, S, E), jnp.float32),
        scratch_shapes=[pltpu.VMEM((S, B * E), jnp.float32)],
    )(x, gate_weight)

    return pl.pallas_call(
        functools.partial(_expert_body, E=E),
        grid=(B, S // _BS),
        in_specs=[
            pl.BlockSpec((1, _BS, D), lambda bi, si: (bi, si, 0)),
            pl.BlockSpec((E, D, D), lambda bi, si: (0, 0, 0)),
            pl.BlockSpec((E, D), lambda bi, si: (0, 0)),
            pl.BlockSpec((1, _BS, E), lambda bi, si: (bi, si, 0)),
        ],
        out_specs=pl.BlockSpec((1, _BS, D), lambda bi, si: (bi, si, 0)),
        out_shape=jax.ShapeDtypeStruct((B, S, D), jnp.float32),
        scratch_shapes=[pltpu.VMEM((E, D, D), jnp.bfloat16)],
    )(x, W, b, g)


# speedup vs baseline: 1.0240x; 1.0069x over previous
import functools

import jax
import jax.numpy as jnp
from jax.experimental import pallas as pl
from jax.experimental.pallas import tpu as pltpu

_CAP = 2
_BS = 512


def _routing_body(x_ref, gw_ref, g_ref, aff_ref, *, K, S, E, B, NC):
    ci = pl.program_id(0)
    SC = S // NC
    for bi in range(B):
        logits = jnp.dot(x_ref[bi], gw_ref[...],
                         preferred_element_type=jnp.float32)
        m = jnp.max(logits, axis=-1, keepdims=True)
        ex = jnp.exp(logits - m)
        aff_c = ex / jnp.sum(ex, axis=-1, keepdims=True)
        aff_ref[pl.ds(ci * SC, SC), bi * E:(bi + 1) * E] = aff_c

    @pl.when(ci == NC - 1)
    def _bisect():
        _routing_epilogue(aff_ref, g_ref, K=K, S=S, E=E, B=B)


def _routing_epilogue(aff_ref, g_ref, *, K, S, E, B):
    aff = aff_ref[...]
    ai = jax.lax.bitcast_convert_type(aff, jnp.int32)
    BE = B * E

    def vstep(_, carry):
        lo, hi = carry
        mid = lo + (hi - lo) // 2
        cnt = jnp.sum((ai >= mid).astype(jnp.int32), axis=0, keepdims=True)
        ge = cnt >= K
        return jnp.where(ge, mid, lo), jnp.where(ge, hi, mid)

    lo0 = jnp.zeros((1, BE), jnp.int32)
    hi0 = jnp.full((1, BE), jnp.int32(0x7F800000))
    T, _ = jax.lax.fori_loop(0, 31, vstep, (lo0, hi0))

    gt = ai > T
    tie = ai == T
    n_gt = jnp.sum(gt.astype(jnp.int32), axis=0, keepdims=True)
    d = K - n_gt

    idx = jax.lax.broadcasted_iota(jnp.int32, (S, BE), 0)

    def istep(_, carry):
        lo2, hi2 = carry
        mid = lo2 + (hi2 - lo2) // 2
        cnt = jnp.sum((tie & (idx <= mid)).astype(jnp.int32),
                      axis=0, keepdims=True)
        ok = cnt >= d
        return jnp.where(ok, lo2, mid), jnp.where(ok, mid, hi2)

    lo2 = jnp.full((1, BE), -1, jnp.int32)
    hi2 = jnp.full((1, BE), S - 1, jnp.int32)
    _, M = jax.lax.fori_loop(0, 12, istep, (lo2, hi2))

    mask = gt | (tie & (idx <= M))
    g = aff * mask.astype(jnp.float32)
    for bi in range(B):
        g_ref[bi] = g[:, bi * E:(bi + 1) * E]


def _expert_body(x_ref, w_ref, b_ref, g_ref, out_ref, wb_ref, *, E):
    si = pl.program_id(0) * pl.num_programs(1) + pl.program_id(1)

    @pl.when(si == 0)
    def _convert_weights():
        for e in range(E):
            wb_ref[e] = w_ref[e].astype(jnp.bfloat16)

    xx = x_ref[0].astype(jnp.bfloat16)
    gg = g_ref[0]
    acc = None
    for e in range(E):
        y = jnp.dot(xx, wb_ref[e],
                    preferred_element_type=jnp.float32)
        y = y + b_ref[e:e + 1, :]
        c = gg[:, e:e + 1] * y
        acc = c if acc is None else acc + c
    out_ref[0] = acc


@jax.jit
def kernel(x, gate_weight, W, b):
    B, S, D = x.shape
    E = gate_weight.shape[1]
    K = (S * _CAP) // E

    NC = 4
    g = pl.pallas_call(
        functools.partial(_routing_body, K=K, S=S, E=E, B=B, NC=NC),
        grid=(NC,),
        in_specs=[
            pl.BlockSpec((B, S // NC, D), lambda i: (0, i, 0)),
            pl.BlockSpec((D, E), lambda i: (0, 0)),
        ],
        out_specs=pl.BlockSpec((B, S, E), lambda i: (0, 0, 0)),
        out_shape=jax.ShapeDtypeStruct((B, S, E), jnp.float32),
        scratch_shapes=[pltpu.VMEM((S, B * E), jnp.float32)],
    )(x, gate_weight)

    return pl.pallas_call(
        functools.partial(_expert_body, E=E),
        grid=(B, S // _BS),
        in_specs=[
            pl.BlockSpec((1, _BS, D), lambda bi, si: (bi, si, 0)),
            pl.BlockSpec((E, D, D), lambda bi, si: (0, 0, 0)),
            pl.BlockSpec((E, D), lambda bi, si: (0, 0)),
            pl.BlockSpec((1, _BS, E), lambda bi, si: (bi, si, 0)),
        ],
        out_specs=pl.BlockSpec((1, _BS, D), lambda bi, si: (bi, si, 0)),
        out_shape=jax.ShapeDtypeStruct((B, S, D), jnp.float32),
        scratch_shapes=[pltpu.VMEM((E, D, D), jnp.bfloat16)],
    )(x, W, b, g)
